# Initial kernel scaffold; baseline (speedup 1.0000x reference)
import functools
import math

import jax
import jax.numpy as jnp
from jax import lax
from jax.experimental import pallas as pl
from jax.experimental.pallas import tpu as pltpu
from jax.experimental.pallas import tpu_sc as plsc

N = 2048
E = 65536
D = 256
EMB = 1024
H = 4
C = 64
EW_W = 16

NC = 2
NS = 16
NW = NC * NS
E_PER_W = E // NW
CH = 128
ROWS_PER_S = N // NS

TE = 1024


def _ln(v, g, b, eps=1e-5):
    mu = jnp.mean(v, axis=-1, keepdims=True)
    var = jnp.mean((v - mu) ** 2, axis=-1, keepdims=True)
    return (v - mu) * jax.lax.rsqrt(var + eps) * g + b


def _q_body(x_ref, wq_ref, bq_ref, q_ref):
    q_ref[...] = (
        jnp.dot(x_ref[...], wq_ref[...], preferred_element_type=jnp.float32)
        + bq_ref[...]
    )


def _q_proj(x, Wq, bq2):
    return pl.pallas_call(
        _q_body,
        out_shape=jax.ShapeDtypeStruct((N, D), jnp.float32),
    )(x, Wq, bq2)


_MESH = plsc.VectorSubcoreMesh(core_axis_name="c", subcore_axis_name="s")


@functools.partial(
    pl.kernel,
    out_type=(
        jax.ShapeDtypeStruct((E, D), jnp.float32),
        jax.ShapeDtypeStruct((E, D), jnp.float32),
    ),
    mesh=_MESH,
    scratch_types=[
        pltpu.VMEM((CH,), jnp.int32),
        pltpu.VMEM((CH, D), jnp.float32),
        pltpu.SemaphoreType.DMA,
    ],
)
def _sc_gather(x_hbm, q_hbm, dst_hbm, src_hbm, xd_hbm, qs_hbm, idx_v, rows_v, sem):
    c = lax.axis_index("c")
    s = lax.axis_index("s")
    base = (c * NS + s) * E_PER_W

    def chunk(i, carry):
        off = base + i * CH
        pltpu.sync_copy(dst_hbm.at[pl.ds(off, CH)], idx_v)
        pltpu.async_copy(x_hbm.at[idx_v], rows_v, sem).wait()
        pltpu.sync_copy(rows_v, xd_hbm.at[pl.ds(off, CH)])
        pltpu.sync_copy(src_hbm.at[pl.ds(off, CH)], idx_v)
        pltpu.async_copy(q_hbm.at[idx_v], rows_v, sem).wait()
        pltpu.sync_copy(rows_v, qs_hbm.at[pl.ds(off, CH)])
        return carry

    lax.fori_loop(0, E_PER_W // CH, chunk, 0)


def _edge_body(xd, qs, ef, wk, bk, wv, bv, we, be, we1, be1, we2, be2,
               g1, b1, g2, b2, wv_out, ew_out, e2_out):
    z = xd[...] * ef[...]
    k = jnp.dot(z, wk[...], preferred_element_type=jnp.float32) + bk[...]
    v = jnp.dot(z, wv[...], preferred_element_type=jnp.float32) + bv[...]
    prod = qs[...] * k
    scale = 1.0 / math.sqrt(C)
    wv_blocks = []
    ew_cols = []
    for h in range(H):
        sh = jnp.sum(prod[:, h * C:(h + 1) * C], axis=1, keepdims=True) * scale
        eh = jnp.exp(sh)
        wv_blocks.append(eh * v[:, h * C:(h + 1) * C])
        ew_cols.append(eh)
    wv_out[...] = jnp.concatenate(wv_blocks, axis=1)
    ew_cols.append(jnp.zeros((TE, EW_W - H), jnp.float32))
    ew_out[...] = jnp.concatenate(ew_cols, axis=1)
    eo = jnp.dot(z, we[...], preferred_element_type=jnp.float32) + be[...]
    e1 = _ln(ef[...] + eo, g1[...], b1[...])
    h1 = jnp.maximum(
        jnp.dot(e1, we1[...], preferred_element_type=jnp.float32) + be1[...], 0.0
    )
    e2 = e1 + jnp.dot(h1, we2[...], preferred_element_type=jnp.float32) + be2[...]
    e2_out[...] = _ln(e2, g2[...], b2[...])


def _edge_call(xd, qs, ef, Wk, bk2, Wv, bv2, We, be_2, We1, be12, We2, be22,
               g1, b1, g2, b2):
    tile = lambda i: (i, 0)
    rep = lambda i: (0, 0)
    eb = pl.BlockSpec((TE, D), tile)
    wspec = pl.BlockSpec((D, D), rep)
    bspec = pl.BlockSpec((1, D), rep)
    bspec_e = pl.BlockSpec((1, EMB), rep)
    return pl.pallas_call(
        _edge_body,
        grid=(E // TE,),
        in_specs=[
            eb, eb, eb,
            wspec, bspec, wspec, bspec, wspec, bspec,
            pl.BlockSpec((D, EMB), rep), bspec_e,
            pl.BlockSpec((EMB, D), rep), bspec,
            bspec, bspec, bspec, bspec,
        ],
        out_specs=[
            eb,
            pl.BlockSpec((TE, EW_W), tile),
            eb,
        ],
        out_shape=[
            jax.ShapeDtypeStruct((E, D), jnp.float32),
            jax.ShapeDtypeStruct((E, EW_W), jnp.float32),
            jax.ShapeDtypeStruct((E, D), jnp.float32),
        ],
    )(xd, qs, ef, Wk, bk2, Wv, bv2, We, be_2, We1, be12, We2, be22,
      g1, b1, g2, b2)


@functools.partial(
    pl.kernel,
    out_type=(
        jax.ShapeDtypeStruct((NC, N, D), jnp.float32),
        jax.ShapeDtypeStruct((NC, N, EW_W), jnp.float32),
    ),
    mesh=_MESH,
    scratch_types=[
        pltpu.VMEM((CH,), jnp.int32),
        pltpu.VMEM((CH, D), jnp.float32),
        pltpu.VMEM((CH, EW_W), jnp.float32),
        pltpu.VMEM_SHARED((N, D), jnp.float32),
        pltpu.VMEM_SHARED((N, EW_W), jnp.float32),
    ],
)
def _sc_scatter(wv_hbm, ew_hbm, src_hbm, znd_hbm, znw_hbm, o_hbm, den_hbm,
                idx_v, rows_v, ewr_v, o_sh, d_sh):
    c = lax.axis_index("c")
    s = lax.axis_index("s")
    base = (c * NS + s) * E_PER_W
    r0 = s * ROWS_PER_S
    pltpu.sync_copy(znd_hbm, rows_v)
    pltpu.sync_copy(znw_hbm, ewr_v)
    pltpu.sync_copy(rows_v, o_sh.at[pl.ds(r0, ROWS_PER_S)])
    pltpu.sync_copy(ewr_v, d_sh.at[pl.ds(r0, ROWS_PER_S)])
    plsc.subcore_barrier()

    def chunk(i, carry):
        off = base + i * CH
        pltpu.sync_copy(src_hbm.at[pl.ds(off, CH)], idx_v)
        pltpu.sync_copy(wv_hbm.at[pl.ds(off, CH)], rows_v)
        pltpu.sync_copy(ew_hbm.at[pl.ds(off, CH)], ewr_v)
        pltpu.sync_copy(rows_v, o_sh.at[idx_v], add=True)
        pltpu.sync_copy(ewr_v, d_sh.at[idx_v], add=True)
        return carry

    lax.fori_loop(0, E_PER_W // CH, chunk, 0)
    plsc.subcore_barrier()
    pltpu.sync_copy(o_sh.at[pl.ds(r0, ROWS_PER_S)], rows_v)
    pltpu.sync_copy(rows_v, o_hbm.at[c, pl.ds(r0, ROWS_PER_S)])
    pltpu.sync_copy(d_sh.at[pl.ds(r0, ROWS_PER_S)], ewr_v)
    pltpu.sync_copy(ewr_v, den_hbm.at[c, pl.ds(r0, ROWS_PER_S)])


def _node_body(x, opart, dpart, wx1, bx1, wx2, bx2, g1, b1, g2, b2, x2_out):
    o = opart[0] + opart[1]
    den = dpart[0] + dpart[1]
    blocks = []
    for h in range(H):
        dh = den[:, h:h + 1]
        dh = jnp.where(dh > 0.0, dh, 1.0)
        blocks.append(o[:, h * C:(h + 1) * C] / dh)
    attn = jnp.concatenate(blocks, axis=1)
    x1 = _ln(x[...] + attn, g1[...], b1[...])
    h1 = jnp.maximum(
        jnp.dot(x1, wx1[...], preferred_element_type=jnp.float32) + bx1[...], 0.0
    )
    x2 = x1 + jnp.dot(h1, wx2[...], preferred_element_type=jnp.float32) + bx2[...]
    x2_out[...] = _ln(x2, g2[...], b2[...])


def _node_call(x, opart, dpart, Wx1, bx12, Wx2, bx22, g1, b1, g2, b2):
    return pl.pallas_call(
        _node_body,
        out_shape=jax.ShapeDtypeStruct((N, D), jnp.float32),
    )(x, opart, dpart, Wx1, bx12, Wx2, bx22, g1, b1, g2, b2)


def kernel(x, edge_index, edge_feat, Wq, bq, Wk, bk, Wv, bv, We, be,
           Wx1, bx1, Wx2, bx2, We1, be1, We2, be2,
           ln1_g, ln1_b, ln2_g, ln2_b):
    src = edge_index[0]
    dst = edge_index[1]
    r = lambda b: b.reshape(1, -1)
    g1, b1 = r(ln1_g), r(ln1_b)
    g2, b2 = r(ln2_g), r(ln2_b)

    q = _q_proj(x, Wq, r(bq))
    xd, qs = _sc_gather(x, q, dst, src)
    wv, ew, e2 = _edge_call(
        xd, qs, edge_feat, Wk, r(bk), Wv, r(bv), We, r(be),
        We1, r(be1), We2, r(be2), g1, b1, g2, b2)
    znd = jnp.zeros((CH, D), jnp.float32)
    znw = jnp.zeros((CH, EW_W), jnp.float32)
    opart, dpart = _sc_scatter(wv, ew, src, znd, znw)
    x2 = _node_call(x, opart, dpart, Wx1, r(bx1), Wx2, r(bx2), g1, b1, g2, b2)
    return (x2, e2)


# speedup vs baseline: 5.6546x; 5.6546x over previous
import functools
import math

import jax
import jax.numpy as jnp
from jax import lax
from jax.experimental import pallas as pl
from jax.experimental.pallas import tpu as pltpu
from jax.experimental.pallas import tpu_sc as plsc

N = 2048
E = 65536
D = 256
EMB = 1024
H = 4
C = 64
EW_W = 16

NC = 2
NS = 16
NW = NC * NS
E_PER_W = E // NW
CH = 128
ROWS_PER_S = N // NS

TE = 1024


def _ln(v, g, b, eps=1e-5):
    mu = jnp.mean(v, axis=-1, keepdims=True)
    var = jnp.mean((v - mu) ** 2, axis=-1, keepdims=True)
    return (v - mu) * jax.lax.rsqrt(var + eps) * g + b


def _q_body(x_ref, wq_ref, bq_ref, q_ref):
    q_ref[...] = (
        jnp.dot(x_ref[...], wq_ref[...], preferred_element_type=jnp.float32)
        + bq_ref[...]
    )


def _q_proj(x, Wq, bq2):
    return pl.pallas_call(
        _q_body,
        out_shape=jax.ShapeDtypeStruct((N, D), jnp.float32),
    )(x, Wq, bq2)


@functools.lru_cache(maxsize=None)
def _make_sc_gather():
    mesh = plsc.VectorSubcoreMesh(core_axis_name="c", subcore_axis_name="s")

    @functools.partial(
        pl.kernel,
        out_type=(
            jax.ShapeDtypeStruct((E, D), jnp.float32),
            jax.ShapeDtypeStruct((E, D), jnp.float32),
        ),
        mesh=mesh,
        scratch_types=[
            pltpu.VMEM((CH,), jnp.int32),
            pltpu.VMEM((CH, D), jnp.float32),
            pltpu.SemaphoreType.DMA,
        ],
    )
    def _sc_gather(x_hbm, q_hbm, dst_hbm, src_hbm, xd_hbm, qs_hbm,
                   idx_v, rows_v, sem):
        c = lax.axis_index("c")
        s = lax.axis_index("s")
        base = (c * NS + s) * E_PER_W

        def chunk(i, carry):
            off = base + i * CH
            pltpu.sync_copy(dst_hbm.at[pl.ds(off, CH)], idx_v)
            pltpu.async_copy(x_hbm.at[idx_v], rows_v, sem).wait()
            pltpu.sync_copy(rows_v, xd_hbm.at[pl.ds(off, CH)])
            pltpu.sync_copy(src_hbm.at[pl.ds(off, CH)], idx_v)
            pltpu.async_copy(q_hbm.at[idx_v], rows_v, sem).wait()
            pltpu.sync_copy(rows_v, qs_hbm.at[pl.ds(off, CH)])
            return carry

        lax.fori_loop(0, E_PER_W // CH, chunk, 0)

    return _sc_gather


NB = 256


def _edge_body(src3, xd, qs, ef, wk, bk, wv, bv, we, be, we1, be1, we2, be2,
               g1, b1, g2, b2, e2_out, o_out, den_out):
    i = pl.program_id(0)
    z = xd[...] * ef[...]
    k = jnp.dot(z, wk[...], preferred_element_type=jnp.float32) + bk[...]
    v = jnp.dot(z, wv[...], preferred_element_type=jnp.float32) + bv[...]
    prod = qs[...] * k
    scale = 1.0 / math.sqrt(C)
    wv_blocks = []
    ew_cols = []
    for h in range(H):
        sh = jnp.sum(prod[:, h * C:(h + 1) * C], axis=1, keepdims=True) * scale
        eh = jnp.exp(sh)
        wv_blocks.append(eh * v[:, h * C:(h + 1) * C])
        ew_cols.append(eh)
    wvals = jnp.concatenate(wv_blocks, axis=1).astype(jnp.bfloat16)
    ew_cols.append(jnp.zeros((TE, EW_W - H), jnp.float32))
    ewts = jnp.concatenate(ew_cols, axis=1).astype(jnp.bfloat16)

    @pl.when(i == 0)
    def _init():
        o_out[...] = jnp.zeros_like(o_out)
        den_out[...] = jnp.zeros_like(den_out)

    srow = src3[0]
    for nb in range(N // NB):
        iota = lax.broadcasted_iota(jnp.int32, (NB, TE), 0) + nb * NB
        oh = (iota == srow).astype(jnp.bfloat16)
        o_out[nb * NB:(nb + 1) * NB, :] += jnp.dot(
            oh, wvals, preferred_element_type=jnp.float32)
        den_out[nb * NB:(nb + 1) * NB, :] += jnp.dot(
            oh, ewts, preferred_element_type=jnp.float32)

    eo = jnp.dot(z, we[...], preferred_element_type=jnp.float32) + be[...]
    e1 = _ln(ef[...] + eo, g1[...], b1[...])
    h1 = jnp.maximum(
        jnp.dot(e1, we1[...], preferred_element_type=jnp.float32) + be1[...], 0.0
    )
    e2 = e1 + jnp.dot(h1, we2[...], preferred_element_type=jnp.float32) + be2[...]
    e2_out[...] = _ln(e2, g2[...], b2[...])


def _edge_call(src3, xd, qs, ef, Wk, bk2, Wv, bv2, We, be_2, We1, be12, We2,
               be22, g1, b1, g2, b2):
    tile = lambda i: (i, 0)
    rep = lambda i: (0, 0)
    eb = pl.BlockSpec((TE, D), tile)
    wspec = pl.BlockSpec((D, D), rep)
    bspec = pl.BlockSpec((1, D), rep)
    bspec_e = pl.BlockSpec((1, EMB), rep)
    return pl.pallas_call(
        _edge_body,
        grid=(E // TE,),
        in_specs=[
            pl.BlockSpec((1, 1, TE), lambda i: (i, 0, 0)),
            eb, eb, eb,
            wspec, bspec, wspec, bspec, wspec, bspec,
            pl.BlockSpec((D, EMB), rep), bspec_e,
            pl.BlockSpec((EMB, D), rep), bspec,
            bspec, bspec, bspec, bspec,
        ],
        out_specs=[
            eb,
            pl.BlockSpec((N, D), rep),
            pl.BlockSpec((N, EW_W), rep),
        ],
        out_shape=[
            jax.ShapeDtypeStruct((E, D), jnp.float32),
            jax.ShapeDtypeStruct((N, D), jnp.float32),
            jax.ShapeDtypeStruct((N, EW_W), jnp.float32),
        ],
    )(src3, xd, qs, ef, Wk, bk2, Wv, bv2, We, be_2, We1, be12, We2, be22,
      g1, b1, g2, b2)


def _node_body(x, o_in, den_in, wx1, bx1, wx2, bx2, g1, b1, g2, b2, x2_out):
    o = o_in[...]
    den = den_in[...]
    blocks = []
    for h in range(H):
        dh = den[:, h:h + 1]
        dh = jnp.where(dh > 0.0, dh, 1.0)
        blocks.append(o[:, h * C:(h + 1) * C] / dh)
    attn = jnp.concatenate(blocks, axis=1)
    x1 = _ln(x[...] + attn, g1[...], b1[...])
    h1 = jnp.maximum(
        jnp.dot(x1, wx1[...], preferred_element_type=jnp.float32) + bx1[...], 0.0
    )
    x2 = x1 + jnp.dot(h1, wx2[...], preferred_element_type=jnp.float32) + bx2[...]
    x2_out[...] = _ln(x2, g2[...], b2[...])


def _node_call(x, o, den, Wx1, bx12, Wx2, bx22, g1, b1, g2, b2):
    return pl.pallas_call(
        _node_body,
        out_shape=jax.ShapeDtypeStruct((N, D), jnp.float32),
    )(x, o, den, Wx1, bx12, Wx2, bx22, g1, b1, g2, b2)


def kernel(x, edge_index, edge_feat, Wq, bq, Wk, bk, Wv, bv, We, be,
           Wx1, bx1, Wx2, bx2, We1, be1, We2, be2,
           ln1_g, ln1_b, ln2_g, ln2_b):
    src = edge_index[0]
    dst = edge_index[1]
    r = lambda b: b.reshape(1, -1)
    g1, b1 = r(ln1_g), r(ln1_b)
    g2, b2 = r(ln2_g), r(ln2_b)

    q = _q_proj(x, Wq, r(bq))
    xd, qs = _make_sc_gather()(x, q, dst, src)
    src3 = src.reshape(E // TE, 1, TE)
    e2, o, den = _edge_call(
        src3, xd, qs, edge_feat, Wk, r(bk), Wv, r(bv), We, r(be),
        We1, r(be1), We2, r(be2), g1, b1, g2, b2)
    x2 = _node_call(x, o, den, Wx1, r(bx1), Wx2, r(bx2), g1, b1, g2, b2)
    return (x2, e2)
